# baseline (speedup 1.0000x reference)
import functools

import jax
import jax.numpy as jnp
from jax import lax
from jax.experimental import pallas as pl
from jax.experimental.pallas import tpu as pltpu
from jax.experimental.pallas import tpu_sc as plsc

NB = 64
NC = 2
NS = 16
NW = NC * NS
TASKS = NB * 16
TPW = TASKS // NW


def _qpool_body(rho_hbm, out_hbm, spm,
                ab0, ab1, cd0, cd1, ob0, ob1,
                sh0, sh1, st0, st1, sc0, sc1, so0, so1):
    cid = lax.axis_index("c")
    sid = lax.axis_index("s")
    wid = sid * NC + cid
    iota = lax.iota(jnp.int32, 16)
    iota2 = iota * 2
    AB, CD, OB = (ab0, ab1), (cd0, cd1), (ob0, ob1)
    SH, ST, SC, SO = (sh0, sh1), (st0, st1), (sc0, sc1), (so0, so1)

    def bp(i):
        t = wid * TPW + i
        return t // 16, t % 16

    def h2s_descr(i, s):
        b, p = bp(i)
        return pltpu.make_async_copy(
            rho_hbm.at[b, 2 * p, :, :], spm.at[sid], SH[s])

    def s2t_descr(i, s):
        return pltpu.make_async_copy(spm.at[sid], AB[s], ST[s])

    def cd_descr(i, s):
        b, p = bp(i)
        cwin = (64 * p + 32) // 128 * 128
        return pltpu.make_async_copy(
            rho_hbm.at[b, 2 * p + 1, :, pl.ds(cwin, 128)], CD[s], SC[s])

    def out_descr(i, s):
        b, p = bp(i)
        return pltpu.make_async_copy(
            OB[s], out_hbm.at[b, pl.ds(p * 16, 16), :], SO[s])

    def compute(i, s):
        in_ab, in_cd, outb = AB[s], CD[s], OB[s]
        _, p = bp(i)
        co = 32 + 64 * (p % 2)
        for q in range(16):
            qs = jnp.full((16,), q, jnp.int32)
            row_a = jnp.full((16,), 2 * q, jnp.int32)
            for r in range(16):
                avec = plsc.load_gather(in_ab, [row_a, iota2 + 64 * r])
                outb[q, pl.ds(r * 16, 16)] = avec
            bvec = plsc.load_gather(
                in_ab, [row_a + 1, iota * 64 + (2 * q + 1)])
            plsc.addupdate_scatter(outb, [qs, iota * 16 + q], bvec)
            cvec = plsc.load_gather(in_cd, [row_a, iota2 + co])
            dvec = plsc.load_gather(
                in_cd, [row_a + 1, jnp.full((16,), 2 * q + 1, jnp.int32) + co])
            cvec = cvec + jnp.where(iota == q, dvec, jnp.zeros((16,), jnp.float32))
            cur = outb[q, pl.ds(p * 16, 16)]
            outb[q, pl.ds(p * 16, 16)] = cur + cvec

    h2s_descr(0, 0).start()
    for s in range(2):
        cd_descr(s, s).start()
    h2s_descr(0, 0).wait()
    s2t_descr(0, 0).start()

    def pair(k, carry):
        for s in range(2):
            i = 2 * k + s
            s2t_descr(i, s).wait()

            @pl.when(i + 1 < TPW)
            def _next_h2s():
                h2s_descr(i + 1, 1 - s).start()

            cd_descr(i, s).wait()

            @pl.when(k > 0)
            def _wait_out():
                out_descr(i - 2, s).wait()

            compute(i, s)
            out_descr(i, s).start()

            @pl.when(i + 2 < TPW)
            def _next_cd():
                cd_descr(i + 2, s).start()

            @pl.when(i + 1 < TPW)
            def _next_s2t():
                h2s_descr(i + 1, 1 - s).wait()
                s2t_descr(i + 1, 1 - s).start()
        return carry

    lax.fori_loop(0, TPW // 2, pair, 0)

    for s in range(2):
        out_descr(TPW - 2 + s, s).wait()


@functools.partial(
    pl.kernel,
    out_type=jax.ShapeDtypeStruct((NB, 256, 256), jnp.float32),
    mesh=plsc.VectorSubcoreMesh(core_axis_name="c", subcore_axis_name="s"),
    scratch_types=[
        pltpu.VMEM_SHARED((NS, 32, 1024), jnp.float32),
        pltpu.VMEM((32, 1024), jnp.float32),
        pltpu.VMEM((32, 1024), jnp.float32),
        pltpu.VMEM((32, 128), jnp.float32),
        pltpu.VMEM((32, 128), jnp.float32),
        pltpu.VMEM((16, 256), jnp.float32),
        pltpu.VMEM((16, 256), jnp.float32),
        pltpu.SemaphoreType.DMA,
        pltpu.SemaphoreType.DMA,
        pltpu.SemaphoreType.DMA,
        pltpu.SemaphoreType.DMA,
        pltpu.SemaphoreType.DMA,
        pltpu.SemaphoreType.DMA,
        pltpu.SemaphoreType.DMA,
        pltpu.SemaphoreType.DMA,
    ],
    compiler_params=pltpu.CompilerParams(
        use_tc_tiling_on_sc=True, needs_layout_passes=False),
)
def _qpool_sc(rho_hbm, out_hbm, spm,
              ab0, ab1, cd0, cd1, ob0, ob1,
              sh0, sh1, st0, st1, sc0, sc1, so0, so1):
    _qpool_body(rho_hbm, out_hbm, spm,
                ab0, ab1, cd0, cd1, ob0, ob1,
                sh0, sh1, st0, st1, sc0, sc1, so0, so1)


def kernel(rho):
    rho3 = rho.reshape(NB, 32, 32, 1024)
    return _qpool_sc(rho3)


# speedup vs baseline: 1.4491x; 1.4491x over previous
import functools

import jax
import jax.numpy as jnp
from jax import lax
from jax.experimental import pallas as pl
from jax.experimental.pallas import tpu as pltpu
from jax.experimental.pallas import tpu_sc as plsc

NB = 64
NC = 2
NS = 16
NW = NC * NS
TASKS = NB * 16
TPW = TASKS // NW
VITER = (TPW + 3) // 3 + 1


def _qpool_body(rho_hbm, out_hbm,
                ab0, ab1, ab2, cd0, cd1, cd2, ob0, ob1, ob2,
                si0, si1, si2, so0, so1, so2):
    wid = lax.axis_index("s") * NC + lax.axis_index("c")
    iota = lax.iota(jnp.int32, 16)
    iota2 = iota * 2
    AB, CD, OB = (ab0, ab1, ab2), (cd0, cd1, cd2), (ob0, ob1, ob2)
    SI, SO = (si0, si1, si2), (so0, so1, so2)

    def bp(i):
        t = wid * TPW + i
        return t // 16, t % 16

    def in_descrs(i, s):
        b, p = bp(i)
        cwin = (64 * p + 32) // 128 * 128
        d1 = pltpu.make_async_copy(rho_hbm.at[b, 2 * p, :, :], AB[s], SI[s])
        d2 = pltpu.make_async_copy(
            rho_hbm.at[b, 2 * p + 1, :, pl.ds(cwin, 128)], CD[s], SI[s])
        return d1, d2

    def out_descr(i, s):
        b, p = bp(i)
        return pltpu.make_async_copy(
            OB[s], out_hbm.at[b, pl.ds(p * 16, 16), :], SO[s])

    def compute(i, s):
        in_ab, in_cd, outb = AB[s], CD[s], OB[s]
        _, p = bp(i)
        co = 32 + 64 * (p % 2)
        for q in range(16):
            qs = jnp.full((16,), q, jnp.int32)
            row_a = jnp.full((16,), 2 * q, jnp.int32)
            for r in range(16):
                avec = plsc.load_gather(in_ab, [row_a, iota2 + 64 * r])
                outb[q, pl.ds(r * 16, 16)] = avec
            bvec = plsc.load_gather(
                in_ab, [row_a + 1, iota * 64 + (2 * q + 1)])
            plsc.addupdate_scatter(outb, [qs, iota * 16 + q], bvec)
            cvec = plsc.load_gather(in_cd, [row_a, iota2 + co])
            dvec = plsc.load_gather(
                in_cd, [row_a + 1, jnp.full((16,), 2 * q + 1, jnp.int32) + co])
            cvec = cvec + jnp.where(iota == q, dvec, jnp.zeros((16,), jnp.float32))
            cur = outb[q, pl.ds(p * 16, 16)]
            outb[q, pl.ds(p * 16, 16)] = cur + cvec

    for s in range(2):
        d1, d2 = in_descrs(s, s)
        d1.start()
        d2.start()

    def triple(k, carry):
        for s in range(3):
            i = 3 * k + s

            @pl.when(i < TPW)
            def _step():
                d1, d2 = in_descrs(i, s)
                d1.wait()
                d2.wait()

                @pl.when(i + 2 < TPW)
                def _prefetch():
                    e1, e2 = in_descrs(i + 2, (s + 2) % 3)
                    e1.start()
                    e2.start()

                @pl.when(i >= 3)
                def _wait_out():
                    out_descr(i - 3, s).wait()

                compute(i, s)
                out_descr(i, s).start()
        return carry

    lax.fori_loop(0, VITER, triple, 0)

    for j in range(3):
        i = TPW - 3 + j
        out_descr(i, i % 3).wait()


@functools.partial(
    pl.kernel,
    out_type=jax.ShapeDtypeStruct((NB, 256, 256), jnp.float32),
    mesh=plsc.VectorSubcoreMesh(core_axis_name="c", subcore_axis_name="s"),
    scratch_types=[
        pltpu.VMEM((32, 1024), jnp.float32),
        pltpu.VMEM((32, 1024), jnp.float32),
        pltpu.VMEM((32, 1024), jnp.float32),
        pltpu.VMEM((32, 128), jnp.float32),
        pltpu.VMEM((32, 128), jnp.float32),
        pltpu.VMEM((32, 128), jnp.float32),
        pltpu.VMEM((16, 256), jnp.float32),
        pltpu.VMEM((16, 256), jnp.float32),
        pltpu.VMEM((16, 256), jnp.float32),
        pltpu.SemaphoreType.DMA,
        pltpu.SemaphoreType.DMA,
        pltpu.SemaphoreType.DMA,
        pltpu.SemaphoreType.DMA,
        pltpu.SemaphoreType.DMA,
        pltpu.SemaphoreType.DMA,
    ],
    compiler_params=pltpu.CompilerParams(
        use_tc_tiling_on_sc=True, needs_layout_passes=False),
)
def _qpool_sc(rho_hbm, out_hbm,
              ab0, ab1, ab2, cd0, cd1, cd2, ob0, ob1, ob2,
              si0, si1, si2, so0, so1, so2):
    _qpool_body(rho_hbm, out_hbm,
                ab0, ab1, ab2, cd0, cd1, cd2, ob0, ob1, ob2,
                si0, si1, si2, so0, so1, so2)


def kernel(rho):
    rho3 = rho.reshape(NB, 32, 32, 1024)
    return _qpool_sc(rho3)
